# Initial kernel scaffold; baseline (speedup 1.0000x reference)
import jax
import jax.numpy as jnp
from jax.experimental import pallas as pl


def kernel(x, edge_index, edge_attr, batch, We1, be1, W11, b11, W12, b12, We2, be2, W21, b21, W22, b22, We3, be3, W31, b31, W32, b32, Wp, bp, Wc, bc):
    raise NotImplementedError("write your pallas kernel here")


# speedup vs baseline: 1.3836x; 1.3836x over previous
import functools

import jax
import jax.numpy as jnp
from jax import lax
from jax.experimental import pallas as pl
from jax.experimental.pallas import tpu as pltpu
from jax.experimental.pallas import tpu_sc as plsc

N = 10000
E = 160000
NG = 64
LANE = 128
NCORE = 2
NTILE = 16
EPT = E // NTILE
KSUB = 80
NSUB = EPT // KSUB
NSEC = 5
SECLEN = NSUB // NSEC
NCHUNK = N // KSUB


def _sc_edge_agg(C):
    CPC = C // NCORE
    mesh = plsc.VectorSubcoreMesh(core_axis_name="c", subcore_axis_name="s",
                                  num_cores=NCORE, num_subcores=NTILE)

    @functools.partial(
        pl.kernel,
        out_type=jax.ShapeDtypeStruct((C, N, LANE), jnp.float32),
        mesh=mesh,
        scratch_types=[
            pltpu.VMEM((SECLEN, KSUB), jnp.int32),
            pltpu.VMEM((SECLEN, KSUB), jnp.int32),
            pltpu.VMEM((KSUB, LANE), jnp.float32),
            pltpu.VMEM((KSUB, LANE), jnp.float32),
            pltpu.VMEM_SHARED((N, LANE), jnp.float32),
            pltpu.SemaphoreType.DMA,
        ],
    )
    def k(xc, ec, src3, dst3, out, src_v, dst_v, m_v, e_v, acc, sem):
        core = lax.axis_index("c")
        tid = lax.axis_index("s")
        nz = -(-NCHUNK // NTILE)
        for ci in range(CPC):
            c = core * CPC + ci

            def zrow(r, carry):
                for u in range(LANE // 16):
                    m_v[r, pl.ds(u * 16, 16)] = jnp.zeros((16,), jnp.float32)
                return carry

            lax.fori_loop(0, KSUB, zrow, 0)
            for z in range(nz):
                zi = tid + z * NTILE

                @pl.when(zi < NCHUNK)
                def _():
                    pltpu.sync_copy(m_v, acc.at[pl.ds(zi * KSUB, KSUB)])
            plsc.subcore_barrier()

            def esec(g, carry):
                pltpu.sync_copy(src3.at[tid * NSEC + g], src_v)
                pltpu.sync_copy(dst3.at[tid * NSEC + g], dst_v)

                def esub(s, carry2):
                    pltpu.async_copy(xc.at[c].at[src_v.at[s]], m_v, sem).wait()
                    ebase = tid * EPT + (g * SECLEN + s) * KSUB
                    pltpu.sync_copy(ec.at[c, pl.ds(ebase, KSUB)], e_v)

                    def crow(r, cc):
                        for u in range(LANE // 16):
                            sl = pl.ds(u * 16, 16)
                            m_v[r, sl] = jnp.maximum(m_v[r, sl] + e_v[r, sl], 0.0)
                        return cc

                    lax.fori_loop(0, KSUB, crow, 0)
                    pltpu.sync_copy(m_v, acc.at[dst_v.at[s]], add=True)
                    return carry2

                lax.fori_loop(0, SECLEN, esub, 0)
                return carry

            lax.fori_loop(0, NSEC, esec, 0)
            plsc.subcore_barrier()
            for z in range(nz):
                zi = tid + z * NTILE

                @pl.when(zi < NCHUNK)
                def _():
                    sl = pl.ds(zi * KSUB, KSUB)
                    pltpu.sync_copy(acc.at[sl], out.at[c, sl])
            plsc.subcore_barrier()

    return k


def _edge_lin3(edge_attr, Wcat, bcat, C1, C2, C3):
    EB = 2000

    def body(ea_ref, w_ref, b_ref, o1, o2, o3):
        t = jnp.dot(ea_ref[...], w_ref[...],
                    preferred_element_type=jnp.float32) + b_ref[...]
        off = 0
        for o_ref, C in ((o1, C1), (o2, C2), (o3, C3)):
            for c in range(C):
                o_ref[c] = t[:, off + c * LANE:off + (c + 1) * LANE]
            off += C * LANE

    Dtot = Wcat.shape[1]
    return pl.pallas_call(
        body,
        grid=(E // EB,),
        in_specs=[
            pl.BlockSpec((EB, 16), lambda i: (i, 0)),
            pl.BlockSpec((16, Dtot), lambda i: (0, 0)),
            pl.BlockSpec((1, Dtot), lambda i: (0, 0)),
        ],
        out_specs=[
            pl.BlockSpec((C1, EB, LANE), lambda i: (0, i, 0)),
            pl.BlockSpec((C2, EB, LANE), lambda i: (0, i, 0)),
            pl.BlockSpec((C3, EB, LANE), lambda i: (0, i, 0)),
        ],
        out_shape=[
            jax.ShapeDtypeStruct((C1, E, LANE), jnp.float32),
            jax.ShapeDtypeStruct((C2, E, LANE), jnp.float32),
            jax.ShapeDtypeStruct((C3, E, LANE), jnp.float32),
        ],
    )(edge_attr, Wcat, bcat)


def _mlp(xc, aggc, Wa, ba, Wb, bb):
    Cin = xc.shape[0]
    H = Wa.shape[1]
    H2 = Wb.shape[1]
    Cout = H2 // LANE
    NB = 1000

    def body(x_ref, a_ref, wa_ref, ba_ref, wb_ref, bb_ref, out_ref):
        wa = wa_ref[...]
        acc = jnp.zeros((NB, H), jnp.float32)
        for c in range(Cin):
            h = x_ref[c] + a_ref[c]
            acc += jnp.dot(h, wa[c * LANE:(c + 1) * LANE, :],
                           preferred_element_type=jnp.float32)
        t = jnp.maximum(acc + ba_ref[...], 0.0)
        wb = wb_ref[...]
        for c in range(Cout):
            o = jnp.dot(t, wb[:, c * LANE:(c + 1) * LANE],
                        preferred_element_type=jnp.float32)
            o = o + bb_ref[:, c * LANE:(c + 1) * LANE]
            out_ref[c] = jnp.where(o > 0.0, o, 0.01 * o)

    Din = Wa.shape[0]
    return pl.pallas_call(
        body,
        grid=(N // NB,),
        in_specs=[
            pl.BlockSpec((Cin, NB, LANE), lambda i: (0, i, 0)),
            pl.BlockSpec((Cin, NB, LANE), lambda i: (0, i, 0)),
            pl.BlockSpec((Din, H), lambda i: (0, 0)),
            pl.BlockSpec((1, H), lambda i: (0, 0)),
            pl.BlockSpec((H, H2), lambda i: (0, 0)),
            pl.BlockSpec((1, H2), lambda i: (0, 0)),
        ],
        out_specs=pl.BlockSpec((Cout, NB, LANE), lambda i: (0, i, 0)),
        out_shape=jax.ShapeDtypeStruct((Cout, N, LANE), jnp.float32),
    )(xc, aggc, Wa, ba, Wb, bb)


def _pool_heads(h3c, batch3, Wc, bc, Wp, bp):
    NB = 1000
    G = N // NB
    CH = h3c.shape[0]

    def body(h_ref, b_ref, wc_ref, bc_ref, wp_ref, bp_ref,
             logits_ref, feat_ref, sums_ref, cnt_ref):
        i = pl.program_id(0)

        @pl.when(i == 0)
        def _():
            sums_ref[...] = jnp.zeros_like(sums_ref)
            cnt_ref[...] = jnp.zeros_like(cnt_ref)

        bvec = b_ref[0, 0]
        onehot = (bvec[:, None] == lax.broadcasted_iota(
            jnp.int32, (NB, NG), 1)).astype(jnp.float32)
        dn = (((0,), (0,)), ((), ()))
        cnt_ref[...] += lax.dot_general(
            onehot, jnp.ones((NB, LANE), jnp.float32), dn,
            preferred_element_type=jnp.float32)
        for c in range(CH):
            sums_ref[:, c * LANE:(c + 1) * LANE] += lax.dot_general(
                onehot, h_ref[c], dn, preferred_element_type=jnp.float32)

        @pl.when(i == G - 1)
        def _():
            cnt = jnp.maximum(cnt_ref[...], 1.0)
            pooled = sums_ref[...] / jnp.concatenate([cnt] * CH, axis=1)
            logits_ref[...] = jnp.dot(
                pooled, wc_ref[...], preferred_element_type=jnp.float32
            ) + bc_ref[...]
            proj = jnp.dot(
                pooled, wp_ref[...], preferred_element_type=jnp.float32
            ) + bp_ref[...]
            nrm = jnp.sqrt(jnp.sum(proj * proj, axis=1, keepdims=True))
            feat_ref[...] = proj / jnp.maximum(nrm, 1e-12)

    H = CH * LANE
    return pl.pallas_call(
        body,
        grid=(G,),
        in_specs=[
            pl.BlockSpec((CH, NB, LANE), lambda i: (0, i, 0)),
            pl.BlockSpec((1, 1, NB), lambda i: (i, 0, 0)),
            pl.BlockSpec((H, NG), lambda i: (0, 0)),
            pl.BlockSpec((1, NG), lambda i: (0, 0)),
            pl.BlockSpec((H, LANE), lambda i: (0, 0)),
            pl.BlockSpec((1, LANE), lambda i: (0, 0)),
        ],
        out_specs=[
            pl.BlockSpec((NG, NG), lambda i: (0, 0)),
            pl.BlockSpec((NG, LANE), lambda i: (0, 0)),
        ],
        out_shape=[
            jax.ShapeDtypeStruct((NG, NG), jnp.float32),
            jax.ShapeDtypeStruct((NG, LANE), jnp.float32),
        ],
        scratch_shapes=[
            pltpu.VMEM((NG, H), jnp.float32),
            pltpu.VMEM((NG, LANE), jnp.float32),
        ],
    )(h3c, batch3, Wc, bc, Wp, bp)


def kernel(x, edge_index, edge_attr, batch,
           We1, be1, W11, b11, W12, b12,
           We2, be2, W21, b21, W22, b22,
           We3, be3, W31, b31, W32, b32,
           Wp, bp, Wc, bc):
    src3 = edge_index[0].reshape(NTILE * NSEC, SECLEN, KSUB)
    dst3 = edge_index[1].reshape(NTILE * NSEC, SECLEN, KSUB)
    batch3 = batch.reshape(N // 1000, 1, 1000)

    DIN = x.shape[1]
    C1 = DIN // LANE
    C2 = We2.shape[1] // LANE
    C3 = We3.shape[1] // LANE

    Wcat = jnp.concatenate([We1, We2, We3], axis=1)
    bcat = jnp.concatenate([be1, be2, be3])[None, :]
    e1, e2, e3 = _edge_lin3(edge_attr, Wcat, bcat, C1, C2, C3)

    xc = x.reshape(N, C1, LANE).transpose(1, 0, 2)

    agg1 = _sc_edge_agg(C1)(xc, e1, src3, dst3)
    h1 = _mlp(xc, agg1, W11, b11[None, :], W12, b12[None, :])
    agg2 = _sc_edge_agg(C2)(h1, e2, src3, dst3)
    h2 = _mlp(h1, agg2, W21, b21[None, :], W22, b22[None, :])
    agg3 = _sc_edge_agg(C3)(h2, e3, src3, dst3)
    h3 = _mlp(h2, agg3, W31, b31[None, :], W32, b32[None, :])

    logits, features = _pool_heads(
        h3, batch3, Wc, bc[None, :], Wp, bp[None, :])
    return (logits, features)


# speedup vs baseline: 1.8352x; 1.3263x over previous
import functools

import jax
import jax.numpy as jnp
from jax import lax
from jax.experimental import pallas as pl
from jax.experimental.pallas import tpu as pltpu
from jax.experimental.pallas import tpu_sc as plsc

N = 10000
E = 160000
NG = 64
LANE = 128
NCORE = 2
NTILE = 16
EPT = E // NTILE
KSUB = 40
NSUB = EPT // KSUB
NSEC = 5
SECLEN = NSUB // NSEC
NPAIR = SECLEN // 2
NCHUNK = N // KSUB


def _sc_edge_agg(C):
    CPC = C // NCORE
    mesh = plsc.VectorSubcoreMesh(core_axis_name="c", subcore_axis_name="s",
                                  num_cores=NCORE, num_subcores=NTILE)

    @functools.partial(
        pl.kernel,
        out_type=jax.ShapeDtypeStruct((C, N, LANE), jnp.float32),
        mesh=mesh,
        scratch_types=[
            pltpu.VMEM((SECLEN, KSUB), jnp.int32),
            pltpu.VMEM((SECLEN, KSUB), jnp.int32),
            pltpu.VMEM((KSUB, LANE), jnp.float32),
            pltpu.VMEM((KSUB, LANE), jnp.float32),
            pltpu.VMEM((KSUB, LANE), jnp.float32),
            pltpu.VMEM((KSUB, LANE), jnp.float32),
            pltpu.VMEM_SHARED((N, LANE), jnp.float32),
            pltpu.SemaphoreType.DMA,
            pltpu.SemaphoreType.DMA,
            pltpu.SemaphoreType.DMA,
            pltpu.SemaphoreType.DMA,
        ],
    )
    def k(xc, ec, src3, dst3, out, src_v, dst_v,
          m_a, m_b, e_a, e_b, acc, gsem_a, gsem_b, ssem_a, ssem_b):
        core = lax.axis_index("c")
        tid = lax.axis_index("s")
        nz = -(-NCHUNK // NTILE)

        def fetch(g, s, m_ref, e_ref, sem):
            gd = pltpu.async_copy(xc.at[c].at[src_v.at[s]], m_ref, sem)
            ebase = tid * EPT + (g * SECLEN + s) * KSUB
            ed = pltpu.async_copy(ec.at[c, pl.ds(ebase, KSUB)], e_ref, sem)
            return gd, ed

        def wait2(m_ref, e_ref, sem):
            pltpu.make_async_copy(xc.at[c, pl.ds(0, KSUB)], m_ref, sem).wait()
            pltpu.make_async_copy(ec.at[c, pl.ds(0, KSUB)], e_ref, sem).wait()

        def compute(m_ref, e_ref):
            def crow(r, cc):
                for rr in range(4):
                    for u in range(LANE // 16):
                        sl = pl.ds(u * 16, 16)
                        row = r * 4 + rr
                        m_ref[row, sl] = jnp.maximum(
                            m_ref[row, sl] + e_ref[row, sl], 0.0)
                return cc

            lax.fori_loop(0, KSUB // 4, crow, 0)

        for ci in range(CPC):
            c = core * CPC + ci

            def zrow(r, carry):
                for u in range(LANE // 16):
                    m_a[r, pl.ds(u * 16, 16)] = jnp.zeros((16,), jnp.float32)
                return carry

            lax.fori_loop(0, KSUB, zrow, 0)
            for z in range(nz):
                zi = tid + z * NTILE

                @pl.when(zi < NCHUNK)
                def _():
                    pltpu.sync_copy(m_a, acc.at[pl.ds(zi * KSUB, KSUB)])
            plsc.subcore_barrier()

            def esec(g, carry):
                pltpu.sync_copy(src3.at[tid * NSEC + g], src_v)
                pltpu.sync_copy(dst3.at[tid * NSEC + g], dst_v)
                fetch(g, 0, m_a, e_a, gsem_a)

                def epair(si, carry2):
                    s0 = si * 2
                    wait2(m_a, e_a, gsem_a)

                    @pl.when(si > 0)
                    def _():
                        pltpu.make_async_copy(
                            m_b, acc.at[dst_v.at[s0]], ssem_b).wait()
                    fetch(g, s0 + 1, m_b, e_b, gsem_b)
                    compute(m_a, e_a)
                    sd_a = pltpu.async_copy(
                        m_a, acc.at[dst_v.at[s0]], ssem_a, add=True)
                    wait2(m_b, e_b, gsem_b)
                    sd_a.wait()

                    @pl.when(si < NPAIR - 1)
                    def _():
                        fetch(g, s0 + 2, m_a, e_a, gsem_a)
                    compute(m_b, e_b)
                    pltpu.async_copy(
                        m_b, acc.at[dst_v.at[s0 + 1]], ssem_b, add=True)
                    return carry2

                lax.fori_loop(0, NPAIR, epair, 0)
                pltpu.make_async_copy(m_b, acc.at[dst_v.at[0]], ssem_b).wait()
                return carry

            lax.fori_loop(0, NSEC, esec, 0)
            plsc.subcore_barrier()
            for z in range(nz):
                zi = tid + z * NTILE

                @pl.when(zi < NCHUNK)
                def _():
                    sl = pl.ds(zi * KSUB, KSUB)
                    pltpu.sync_copy(acc.at[sl], out.at[c, sl])
            plsc.subcore_barrier()

    return k


def _edge_lin3(edge_attr, Wcat, bcat, C1, C2, C3):
    EB = 2000

    def body(ea_ref, w_ref, b_ref, o1, o2, o3):
        t = jnp.dot(ea_ref[...], w_ref[...],
                    preferred_element_type=jnp.float32) + b_ref[...]
        off = 0
        for o_ref, C in ((o1, C1), (o2, C2), (o3, C3)):
            for c in range(C):
                o_ref[c] = t[:, off + c * LANE:off + (c + 1) * LANE]
            off += C * LANE

    Dtot = Wcat.shape[1]
    return pl.pallas_call(
        body,
        grid=(E // EB,),
        in_specs=[
            pl.BlockSpec((EB, 16), lambda i: (i, 0)),
            pl.BlockSpec((16, Dtot), lambda i: (0, 0)),
            pl.BlockSpec((1, Dtot), lambda i: (0, 0)),
        ],
        out_specs=[
            pl.BlockSpec((C1, EB, LANE), lambda i: (0, i, 0)),
            pl.BlockSpec((C2, EB, LANE), lambda i: (0, i, 0)),
            pl.BlockSpec((C3, EB, LANE), lambda i: (0, i, 0)),
        ],
        out_shape=[
            jax.ShapeDtypeStruct((C1, E, LANE), jnp.float32),
            jax.ShapeDtypeStruct((C2, E, LANE), jnp.float32),
            jax.ShapeDtypeStruct((C3, E, LANE), jnp.float32),
        ],
    )(edge_attr, Wcat, bcat)


def _mlp(xc, aggc, Wa, ba, Wb, bb):
    Cin = xc.shape[0]
    H = Wa.shape[1]
    H2 = Wb.shape[1]
    Cout = H2 // LANE
    NB = 1000

    def body(x_ref, a_ref, wa_ref, ba_ref, wb_ref, bb_ref, out_ref):
        wa = wa_ref[...]
        acc = jnp.zeros((NB, H), jnp.float32)
        for c in range(Cin):
            h = x_ref[c] + a_ref[c]
            acc += jnp.dot(h, wa[c * LANE:(c + 1) * LANE, :],
                           preferred_element_type=jnp.float32)
        t = jnp.maximum(acc + ba_ref[...], 0.0)
        wb = wb_ref[...]
        for c in range(Cout):
            o = jnp.dot(t, wb[:, c * LANE:(c + 1) * LANE],
                        preferred_element_type=jnp.float32)
            o = o + bb_ref[:, c * LANE:(c + 1) * LANE]
            out_ref[c] = jnp.where(o > 0.0, o, 0.01 * o)

    Din = Wa.shape[0]
    return pl.pallas_call(
        body,
        grid=(N // NB,),
        in_specs=[
            pl.BlockSpec((Cin, NB, LANE), lambda i: (0, i, 0)),
            pl.BlockSpec((Cin, NB, LANE), lambda i: (0, i, 0)),
            pl.BlockSpec((Din, H), lambda i: (0, 0)),
            pl.BlockSpec((1, H), lambda i: (0, 0)),
            pl.BlockSpec((H, H2), lambda i: (0, 0)),
            pl.BlockSpec((1, H2), lambda i: (0, 0)),
        ],
        out_specs=pl.BlockSpec((Cout, NB, LANE), lambda i: (0, i, 0)),
        out_shape=jax.ShapeDtypeStruct((Cout, N, LANE), jnp.float32),
    )(xc, aggc, Wa, ba, Wb, bb)


def _pool_heads(h3c, batch3, Wc, bc, Wp, bp):
    NB = 1000
    G = N // NB
    CH = h3c.shape[0]

    def body(h_ref, b_ref, wc_ref, bc_ref, wp_ref, bp_ref,
             logits_ref, feat_ref, sums_ref, cnt_ref):
        i = pl.program_id(0)

        @pl.when(i == 0)
        def _():
            sums_ref[...] = jnp.zeros_like(sums_ref)
            cnt_ref[...] = jnp.zeros_like(cnt_ref)

        bvec = b_ref[0, 0]
        onehot = (bvec[:, None] == lax.broadcasted_iota(
            jnp.int32, (NB, NG), 1)).astype(jnp.float32)
        dn = (((0,), (0,)), ((), ()))
        cnt_ref[...] += lax.dot_general(
            onehot, jnp.ones((NB, LANE), jnp.float32), dn,
            preferred_element_type=jnp.float32)
        for c in range(CH):
            sums_ref[:, c * LANE:(c + 1) * LANE] += lax.dot_general(
                onehot, h_ref[c], dn, preferred_element_type=jnp.float32)

        @pl.when(i == G - 1)
        def _():
            cnt = jnp.maximum(cnt_ref[...], 1.0)
            pooled = sums_ref[...] / jnp.concatenate([cnt] * CH, axis=1)
            logits_ref[...] = jnp.dot(
                pooled, wc_ref[...], preferred_element_type=jnp.float32
            ) + bc_ref[...]
            proj = jnp.dot(
                pooled, wp_ref[...], preferred_element_type=jnp.float32
            ) + bp_ref[...]
            nrm = jnp.sqrt(jnp.sum(proj * proj, axis=1, keepdims=True))
            feat_ref[...] = proj / jnp.maximum(nrm, 1e-12)

    H = CH * LANE
    return pl.pallas_call(
        body,
        grid=(G,),
        in_specs=[
            pl.BlockSpec((CH, NB, LANE), lambda i: (0, i, 0)),
            pl.BlockSpec((1, 1, NB), lambda i: (i, 0, 0)),
            pl.BlockSpec((H, NG), lambda i: (0, 0)),
            pl.BlockSpec((1, NG), lambda i: (0, 0)),
            pl.BlockSpec((H, LANE), lambda i: (0, 0)),
            pl.BlockSpec((1, LANE), lambda i: (0, 0)),
        ],
        out_specs=[
            pl.BlockSpec((NG, NG), lambda i: (0, 0)),
            pl.BlockSpec((NG, LANE), lambda i: (0, 0)),
        ],
        out_shape=[
            jax.ShapeDtypeStruct((NG, NG), jnp.float32),
            jax.ShapeDtypeStruct((NG, LANE), jnp.float32),
        ],
        scratch_shapes=[
            pltpu.VMEM((NG, H), jnp.float32),
            pltpu.VMEM((NG, LANE), jnp.float32),
        ],
    )(h3c, batch3, Wc, bc, Wp, bp)


def kernel(x, edge_index, edge_attr, batch,
           We1, be1, W11, b11, W12, b12,
           We2, be2, W21, b21, W22, b22,
           We3, be3, W31, b31, W32, b32,
           Wp, bp, Wc, bc):
    src3 = edge_index[0].reshape(NTILE * NSEC, SECLEN, KSUB)
    dst3 = edge_index[1].reshape(NTILE * NSEC, SECLEN, KSUB)
    batch3 = batch.reshape(N // 1000, 1, 1000)

    DIN = x.shape[1]
    C1 = DIN // LANE
    C2 = We2.shape[1] // LANE
    C3 = We3.shape[1] // LANE

    Wcat = jnp.concatenate([We1, We2, We3], axis=1)
    bcat = jnp.concatenate([be1, be2, be3])[None, :]
    e1, e2, e3 = _edge_lin3(edge_attr, Wcat, bcat, C1, C2, C3)

    xc = x.reshape(N, C1, LANE).transpose(1, 0, 2)

    agg1 = _sc_edge_agg(C1)(xc, e1, src3, dst3)
    h1 = _mlp(xc, agg1, W11, b11[None, :], W12, b12[None, :])
    agg2 = _sc_edge_agg(C2)(h1, e2, src3, dst3)
    h2 = _mlp(h1, agg2, W21, b21[None, :], W22, b22[None, :])
    agg3 = _sc_edge_agg(C3)(h2, e3, src3, dst3)
    h3 = _mlp(h2, agg3, W31, b31[None, :], W32, b32[None, :])

    logits, features = _pool_heads(
        h3, batch3, Wc, bc[None, :], Wp, bp[None, :])
    return (logits, features)


# speedup vs baseline: 2.1404x; 1.1663x over previous
import functools

import jax
import jax.numpy as jnp
from jax import lax
from jax.experimental import pallas as pl
from jax.experimental.pallas import tpu as pltpu
from jax.experimental.pallas import tpu_sc as plsc

N = 10000
E = 160000
NG = 64
LANE = 128
NCORE = 2
NTILE = 16
EPT = E // NTILE
KSUB = 40
NSUB = EPT // KSUB
NSEC = 5
SECLEN = NSUB // NSEC
NPAIR = SECLEN // 2
NCHUNK = N // KSUB


def _sc_edge_agg(C):
    CPC = C // NCORE
    mesh = plsc.VectorSubcoreMesh(core_axis_name="c", subcore_axis_name="s",
                                  num_cores=NCORE, num_subcores=NTILE)

    @functools.partial(
        pl.kernel,
        out_type=jax.ShapeDtypeStruct((C, N, LANE), jnp.float32),
        mesh=mesh,
        scratch_types=[
            pltpu.VMEM((SECLEN, KSUB), jnp.int32),
            pltpu.VMEM((SECLEN, KSUB), jnp.int32),
            pltpu.VMEM((KSUB, LANE), jnp.float32),
            pltpu.VMEM((KSUB, LANE), jnp.float32),
            pltpu.VMEM((KSUB, LANE), jnp.bfloat16),
            pltpu.VMEM((KSUB, LANE), jnp.bfloat16),
            pltpu.VMEM((KSUB, LANE), jnp.float32),
            pltpu.VMEM((KSUB, LANE), jnp.float32),
            pltpu.VMEM_SHARED((N, LANE), jnp.float32),
            pltpu.SemaphoreType.DMA,
            pltpu.SemaphoreType.DMA,
            pltpu.SemaphoreType.DMA,
            pltpu.SemaphoreType.DMA,
        ],
    )
    def k(xc, ec, src3, dst3, out, src_v, dst_v,
          m_a, m_b, e_a, e_b, f_a, f_b, acc, gsem_a, gsem_b, ssem_a, ssem_b):
        core = lax.axis_index("c")
        tid = lax.axis_index("s")
        nz = -(-NCHUNK // NTILE)

        def fetch(g, s, m_ref, e_ref, sem):
            gd = pltpu.async_copy(xc.at[c].at[src_v.at[s]], m_ref, sem)
            ebase = tid * EPT + (g * SECLEN + s) * KSUB
            ed = pltpu.async_copy(ec.at[c, pl.ds(ebase, KSUB)], e_ref, sem)
            return gd, ed

        def wait2(m_ref, e_ref, sem):
            pltpu.make_async_copy(xc.at[c, pl.ds(0, KSUB)], m_ref, sem).wait()
            pltpu.make_async_copy(ec.at[c, pl.ds(0, KSUB)], e_ref, sem).wait()

        def compute(m_ref, e_ref, f_ref):
            def cslice(u, cc):
                sl = pl.ds(u * 16, 16)
                for row in range(KSUB):
                    v = m_ref[row, sl] + e_ref[row, sl].astype(jnp.float32)
                    f_ref[row, sl] = jnp.maximum(v, 0.0)
                return cc

            lax.fori_loop(0, LANE // 16, cslice, 0)

        for ci in range(CPC):
            c = core * CPC + ci

            def zrow(r, carry):
                for u in range(LANE // 16):
                    f_a[r, pl.ds(u * 16, 16)] = jnp.zeros((16,), jnp.float32)
                return carry

            lax.fori_loop(0, KSUB, zrow, 0)
            for z in range(nz):
                zi = tid + z * NTILE

                @pl.when(zi < NCHUNK)
                def _():
                    pltpu.sync_copy(f_a, acc.at[pl.ds(zi * KSUB, KSUB)])
            plsc.subcore_barrier()

            def esec(g, carry):
                pltpu.sync_copy(src3.at[tid * NSEC + g], src_v)
                pltpu.sync_copy(dst3.at[tid * NSEC + g], dst_v)
                fetch(g, 0, m_a, e_a, gsem_a)

                def epair(si, carry2):
                    s0 = si * 2
                    wait2(m_a, e_a, gsem_a)
                    fetch(g, s0 + 1, m_b, e_b, gsem_b)
                    compute(m_a, e_a, f_a)
                    sd_a = pltpu.async_copy(
                        f_a, acc.at[dst_v.at[s0]], ssem_a, add=True)
                    wait2(m_b, e_b, gsem_b)
                    sd_a.wait()

                    @pl.when(si < NPAIR - 1)
                    def _():
                        fetch(g, s0 + 2, m_a, e_a, gsem_a)

                    @pl.when(si > 0)
                    def _():
                        pltpu.make_async_copy(
                            f_b, acc.at[dst_v.at[s0]], ssem_b).wait()
                    compute(m_b, e_b, f_b)
                    pltpu.async_copy(
                        f_b, acc.at[dst_v.at[s0 + 1]], ssem_b, add=True)
                    return carry2

                lax.fori_loop(0, NPAIR, epair, 0)
                pltpu.make_async_copy(f_b, acc.at[dst_v.at[0]], ssem_b).wait()
                return carry

            lax.fori_loop(0, NSEC, esec, 0)
            plsc.subcore_barrier()
            for z in range(nz):
                zi = tid + z * NTILE

                @pl.when(zi < NCHUNK)
                def _():
                    sl = pl.ds(zi * KSUB, KSUB)
                    pltpu.sync_copy(acc.at[sl], out.at[c, sl])
            plsc.subcore_barrier()

    return k


def _edge_lin(edge_attr, Wcat, bcat, Cs):
    EB = 2000

    def body(ea_ref, w_ref, b_ref, *outs):
        t = jnp.dot(ea_ref[...].astype(jnp.bfloat16),
                    w_ref[...].astype(jnp.bfloat16),
                    preferred_element_type=jnp.float32) + b_ref[...]
        off = 0
        for o_ref, C in zip(outs, Cs):
            for c in range(C):
                o_ref[c] = t[:, off + c * LANE:off + (c + 1) * LANE
                             ].astype(jnp.bfloat16)
            off += C * LANE

    Dtot = Wcat.shape[1]
    out = pl.pallas_call(
        body,
        grid=(E // EB,),
        in_specs=[
            pl.BlockSpec((EB, 16), lambda i: (i, 0)),
            pl.BlockSpec((16, Dtot), lambda i: (0, 0)),
            pl.BlockSpec((1, Dtot), lambda i: (0, 0)),
        ],
        out_specs=[
            pl.BlockSpec((C, EB, LANE), lambda i: (0, i, 0)) for C in Cs
        ],
        out_shape=[
            jax.ShapeDtypeStruct((C, E, LANE), jnp.bfloat16) for C in Cs
        ],
    )(edge_attr, Wcat, bcat)
    return out


def _mlp(xc, aggc, Wa, ba, Wb, bb):
    Cin = xc.shape[0]
    H = Wa.shape[1]
    H2 = Wb.shape[1]
    Cout = H2 // LANE
    NB = 1000

    def body(x_ref, a_ref, wa_ref, ba_ref, wb_ref, bb_ref, out_ref):
        wa = wa_ref[...]
        acc = jnp.zeros((NB, H), jnp.float32)
        for c in range(Cin):
            h = x_ref[c].astype(jnp.float32) + a_ref[c]
            acc += jnp.dot(h, wa[c * LANE:(c + 1) * LANE, :],
                           preferred_element_type=jnp.float32)
        t = jnp.maximum(acc + ba_ref[...], 0.0)
        wb = wb_ref[...]
        for c in range(Cout):
            o = jnp.dot(t, wb[:, c * LANE:(c + 1) * LANE],
                        preferred_element_type=jnp.float32)
            o = o + bb_ref[:, c * LANE:(c + 1) * LANE]
            out_ref[c] = jnp.where(o > 0.0, o, 0.01 * o)

    Din = Wa.shape[0]
    return pl.pallas_call(
        body,
        grid=(N // NB,),
        in_specs=[
            pl.BlockSpec((Cin, NB, LANE), lambda i: (0, i, 0)),
            pl.BlockSpec((Cin, NB, LANE), lambda i: (0, i, 0)),
            pl.BlockSpec((Din, H), lambda i: (0, 0)),
            pl.BlockSpec((1, H), lambda i: (0, 0)),
            pl.BlockSpec((H, H2), lambda i: (0, 0)),
            pl.BlockSpec((1, H2), lambda i: (0, 0)),
        ],
        out_specs=pl.BlockSpec((Cout, NB, LANE), lambda i: (0, i, 0)),
        out_shape=jax.ShapeDtypeStruct((Cout, N, LANE), jnp.float32),
    )(xc, aggc, Wa, ba, Wb, bb)


def _pool_heads(h3c, batch3, Wc, bc, Wp, bp):
    NB = 1000
    G = N // NB
    CH = h3c.shape[0]

    def body(h_ref, b_ref, wc_ref, bc_ref, wp_ref, bp_ref,
             logits_ref, feat_ref, sums_ref, cnt_ref):
        i = pl.program_id(0)

        @pl.when(i == 0)
        def _():
            sums_ref[...] = jnp.zeros_like(sums_ref)
            cnt_ref[...] = jnp.zeros_like(cnt_ref)

        bvec = b_ref[0, 0]
        onehot = (bvec[:, None] == lax.broadcasted_iota(
            jnp.int32, (NB, NG), 1)).astype(jnp.float32)
        dn = (((0,), (0,)), ((), ()))
        cnt_ref[...] += lax.dot_general(
            onehot, jnp.ones((NB, LANE), jnp.float32), dn,
            preferred_element_type=jnp.float32)
        for c in range(CH):
            sums_ref[:, c * LANE:(c + 1) * LANE] += lax.dot_general(
                onehot, h_ref[c].astype(jnp.float32), dn,
                preferred_element_type=jnp.float32)

        @pl.when(i == G - 1)
        def _():
            cnt = jnp.maximum(cnt_ref[...], 1.0)
            pooled = sums_ref[...] / jnp.concatenate([cnt] * CH, axis=1)
            logits_ref[...] = jnp.dot(
                pooled, wc_ref[...], preferred_element_type=jnp.float32
            ) + bc_ref[...]
            proj = jnp.dot(
                pooled, wp_ref[...], preferred_element_type=jnp.float32
            ) + bp_ref[...]
            nrm = jnp.sqrt(jnp.sum(proj * proj, axis=1, keepdims=True))
            feat_ref[...] = proj / jnp.maximum(nrm, 1e-12)

    H = CH * LANE
    return pl.pallas_call(
        body,
        grid=(G,),
        in_specs=[
            pl.BlockSpec((CH, NB, LANE), lambda i: (0, i, 0)),
            pl.BlockSpec((1, 1, NB), lambda i: (i, 0, 0)),
            pl.BlockSpec((H, NG), lambda i: (0, 0)),
            pl.BlockSpec((1, NG), lambda i: (0, 0)),
            pl.BlockSpec((H, LANE), lambda i: (0, 0)),
            pl.BlockSpec((1, LANE), lambda i: (0, 0)),
        ],
        out_specs=[
            pl.BlockSpec((NG, NG), lambda i: (0, 0)),
            pl.BlockSpec((NG, LANE), lambda i: (0, 0)),
        ],
        out_shape=[
            jax.ShapeDtypeStruct((NG, NG), jnp.float32),
            jax.ShapeDtypeStruct((NG, LANE), jnp.float32),
        ],
        scratch_shapes=[
            pltpu.VMEM((NG, H), jnp.float32),
            pltpu.VMEM((NG, LANE), jnp.float32),
        ],
    )(h3c, batch3, Wc, bc, Wp, bp)


def kernel(x, edge_index, edge_attr, batch,
           We1, be1, W11, b11, W12, b12,
           We2, be2, W21, b21, W22, b22,
           We3, be3, W31, b31, W32, b32,
           Wp, bp, Wc, bc):
    src3 = edge_index[0].reshape(NTILE * NSEC, SECLEN, KSUB)
    dst3 = edge_index[1].reshape(NTILE * NSEC, SECLEN, KSUB)
    batch3 = batch.reshape(N // 1000, 1, 1000)

    DIN = x.shape[1]
    C1 = DIN // LANE
    C2 = We2.shape[1] // LANE
    C3 = We3.shape[1] // LANE

    (e1,) = _edge_lin(edge_attr, We1, be1[None, :], (C1,))
    e2, e3 = _edge_lin(
        edge_attr, jnp.concatenate([We2, We3], axis=1),
        jnp.concatenate([be2, be3])[None, :], (C2, C3))

    xc = x.reshape(N, C1, LANE).transpose(1, 0, 2)

    agg1 = _sc_edge_agg(C1)(xc, e1, src3, dst3)
    h1 = _mlp(xc, agg1, W11, b11[None, :], W12, b12[None, :])
    agg2 = _sc_edge_agg(C2)(h1, e2, src3, dst3)
    h2 = _mlp(h1, agg2, W21, b21[None, :], W22, b22[None, :])
    agg3 = _sc_edge_agg(C3)(h2, e3, src3, dst3)
    h3 = _mlp(h2, agg3, W31, b31[None, :], W32, b32[None, :])

    logits, features = _pool_heads(
        h3, batch3, Wc, bc[None, :], Wp, bp[None, :])
    return (logits, features)
